# Initial kernel scaffold; baseline (speedup 1.0000x reference)
import jax
import jax.numpy as jnp
from jax.experimental import pallas as pl


def kernel(x, edge_index, edge_attr, hidden, params):
    raise NotImplementedError("write your pallas kernel here")


# speedup vs baseline: 2.0560x; 2.0560x over previous
import functools

import jax
import jax.numpy as jnp
from jax import lax
from jax.experimental import pallas as pl
from jax.experimental.pallas import tpu as pltpu
from jax.experimental.pallas import tpu_sc as plsc

F32 = jnp.float32

N = 10000
E = 320000
F = 128
FE = 16
H = 256
R = 128
L = 64
O = 128

NC = 2
NS = 16
NW = NC * NS
EPW = E // NW
C = 80
CPW = EPW // C
NP = 10240
NPT = NP // NS
WS = 128

_SC_MESH = plsc.VectorSubcoreMesh(core_axis_name="c", subcore_axis_name="s")


def _dot(a, b):
    return jnp.dot(a, b, preferred_element_type=F32)


@functools.partial(
    pl.kernel,
    out_type=(
        jax.ShapeDtypeStruct((E, H), F32),
        jax.ShapeDtypeStruct((E, H), F32),
    ),
    mesh=_SC_MESH,
    scratch_types=[
        pltpu.VMEM((CPW, C), jnp.int32),
        pltpu.VMEM((CPW, C), jnp.int32),
        pltpu.VMEM((C, H), F32),
        pltpu.VMEM((C, H), F32),
    ],
)
def _sc_gather2(a_hbm, b_hbm, row_hbm, col_hbm, oa_hbm, ob_hbm,
                ridx, cidx, bufa, bufb):
    wid = lax.axis_index("s") * NC + lax.axis_index("c")
    pltpu.sync_copy(row_hbm.at[wid], ridx)
    pltpu.sync_copy(col_hbm.at[wid], cidx)

    def step(j, _):
        pltpu.sync_copy(a_hbm.at[ridx.at[j]], bufa)
        pltpu.sync_copy(b_hbm.at[cidx.at[j]], bufb)
        pltpu.sync_copy(bufa, oa_hbm.at[pl.ds(wid * EPW + j * C, C)])
        pltpu.sync_copy(bufb, ob_hbm.at[pl.ds(wid * EPW + j * C, C)])
        return 0

    lax.fori_loop(0, CPW, step, 0)


def _make_sc_scatter(width):
    assert width == WS

    @functools.partial(
        pl.kernel,
        out_type=jax.ShapeDtypeStruct((NC * NP, width), F32),
        mesh=_SC_MESH,
        scratch_types=[
            pltpu.VMEM((EPW,), jnp.int32),
            pltpu.VMEM((C, width), F32),
            pltpu.VMEM_SHARED((NP, width), F32),
        ],
    )
    def scatter(vals_hbm, colf_hbm, zs_hbm, sum_out, cidx, vals, sum_sh):
        cid = lax.axis_index("c")
        sid = lax.axis_index("s")
        wid = sid * NC + cid
        pltpu.sync_copy(zs_hbm.at[pl.ds(0, C)], vals)

        def zstep(t, _):
            pltpu.sync_copy(vals, sum_sh.at[pl.ds(sid * NPT + t * C, C)])
            return 0

        lax.fori_loop(0, NPT // C, zstep, 0)
        pltpu.sync_copy(colf_hbm.at[pl.ds(wid * EPW, EPW)], cidx)
        plsc.subcore_barrier()

        def step(j, _):
            pltpu.sync_copy(vals_hbm.at[pl.ds(wid * EPW + j * C, C)], vals)

            def sub(k, _2):
                idx16 = cidx[pl.ds(j * C + k * 16, 16)]
                pltpu.sync_copy(vals.at[pl.ds(k * 16, 16)],
                                sum_sh.at[idx16], add=True)
                return 0

            lax.fori_loop(0, C // 16, sub, 0)
            return 0

        lax.fori_loop(0, CPW, step, 0)
        plsc.subcore_barrier()

        def wstep(t, _):
            pltpu.sync_copy(sum_sh.at[pl.ds(sid * NPT + t * C, C)], vals)
            pltpu.sync_copy(vals,
                            sum_out.at[pl.ds(cid * NP + sid * NPT + t * C, C)])
            return 0

        lax.fori_loop(0, NPT // C, wstep, 0)

    return scatter


_sc_scatter = _make_sc_scatter(WS)


RB = 2000
EB = 4000


def _full(shape):
    return pl.BlockSpec(shape, lambda i: (0,) * len(shape))


def _k0_body(x_r, wsx_r, wdx_r, b1_r, a_r, b_r):
    x = x_r[...]
    a_r[...] = _dot(x, wsx_r[...]) + b1_r[...]
    b_r[...] = _dot(x, wdx_r[...])


def _k0(x, wsxT, wdxT, b1):
    return pl.pallas_call(
        _k0_body,
        grid=(N // RB,),
        in_specs=[
            pl.BlockSpec((RB, F), lambda i: (i, 0)),
            _full((F, H)), _full((F, H)), _full((1, H)),
        ],
        out_specs=[pl.BlockSpec((RB, H), lambda i: (i, 0))] * 2,
        out_shape=[jax.ShapeDtypeStruct((N, H), F32)] * 2,
    )(x, wsxT, wdxT, b1)


def _k2_body(ga_r, gb_r, ea_r, weT_r, w2T_r, b2_r, out_r):
    h = jax.nn.relu(ga_r[...] + gb_r[...] + _dot(ea_r[...], weT_r[...]))
    msg = _dot(h, w2T_r[...]) + b2_r[...]
    out_r[...] = jnp.concatenate(
        [msg, jnp.ones((EB, FE), F32), jnp.zeros((EB, WS - L - FE), F32)],
        axis=1)


def _k2(ga, gb, ea, weT, w2T, b2):
    return pl.pallas_call(
        _k2_body,
        grid=(E // EB,),
        in_specs=[
            pl.BlockSpec((EB, H), lambda i: (i, 0)),
            pl.BlockSpec((EB, H), lambda i: (i, 0)),
            pl.BlockSpec((EB, FE), lambda i: (i, 0)),
            _full((FE, H)), _full((H, L)), _full((1, L)),
        ],
        out_specs=pl.BlockSpec((EB, WS), lambda i: (i, 0)),
        out_shape=jax.ShapeDtypeStruct((E, WS), F32),
    )(ga, gb, ea, weT, w2T, b2)


def _k6_body(ga_r, gb_r, ea_r, ea1_r, weaT_r, wea1T_r, w2T_r, b2_r, out_r):
    ea1 = ea1_r[...][:, :L]
    h = jax.nn.relu(ga_r[...] + gb_r[...] + _dot(ea_r[...], weaT_r[...])
                    + _dot(ea1, wea1T_r[...]))
    msg = _dot(h, w2T_r[...]) + b2_r[...]
    out_r[...] = jnp.concatenate([msg, jnp.zeros((EB, WS - L), F32)], axis=1)


def _k6(ga, gb, ea, ea1, weaT, wea1T, w2T, b2):
    return pl.pallas_call(
        _k6_body,
        grid=(E // EB,),
        in_specs=[
            pl.BlockSpec((EB, H), lambda i: (i, 0)),
            pl.BlockSpec((EB, H), lambda i: (i, 0)),
            pl.BlockSpec((EB, FE), lambda i: (i, 0)),
            pl.BlockSpec((EB, WS), lambda i: (i, 0)),
            _full((FE, H)), _full((L, H)), _full((H, L)), _full((1, L)),
        ],
        out_specs=pl.BlockSpec((EB, WS), lambda i: (i, 0)),
        out_shape=jax.ShapeDtypeStruct((E, WS), F32),
    )(ga, gb, ea, ea1, weaT, wea1T, w2T, b2)


def _k4_body(x_r, h0_r, sums_r, wihx_r, wiha_r, bih_r, whh_r, bhh_r,
             w2sx_r, w2sh_r, b2s_r, w2dx_r, w2dh_r,
             h1_r, a2_r, b2_r):
    s = sums_r[0] + sums_r[1]
    agg = s[:, :L] / jnp.maximum(s[:, L:L + 1], 1.0)
    x = x_r[...]
    h0 = h0_r[...]
    gi = _dot(x, wihx_r[...]) + _dot(agg, wiha_r[...]) + bih_r[...]
    gh = _dot(h0, whh_r[...]) + bhh_r[...]
    r = jax.nn.sigmoid(gi[:, :R] + gh[:, :R])
    z = jax.nn.sigmoid(gi[:, R:2 * R] + gh[:, R:2 * R])
    n = jnp.tanh(gi[:, 2 * R:] + r * gh[:, 2 * R:])
    h1 = (1.0 - z) * n + z * h0
    h1_r[...] = h1
    a2_r[...] = _dot(x, w2sx_r[...]) + _dot(h1, w2sh_r[...]) + b2s_r[...]
    b2_r[...] = _dot(x, w2dx_r[...]) + _dot(h1, w2dh_r[...])


def _k4(x, h0, sums, wihxT, wihaT, bih, whhT, bhh,
        w2sxT, w2shT, b2s, w2dxT, w2dhT):
    return pl.pallas_call(
        _k4_body,
        grid=(N // RB,),
        in_specs=[
            pl.BlockSpec((RB, F), lambda i: (i, 0)),
            pl.BlockSpec((RB, R), lambda i: (i, 0)),
            pl.BlockSpec((NC, RB, WS), lambda i: (0, i, 0)),
            _full((F, 3 * R)), _full((L, 3 * R)), _full((1, 3 * R)),
            _full((R, 3 * R)), _full((1, 3 * R)),
            _full((F, H)), _full((R, H)), _full((1, H)),
            _full((F, H)), _full((R, H)),
        ],
        out_specs=[
            pl.BlockSpec((RB, R), lambda i: (i, 0)),
            pl.BlockSpec((RB, H), lambda i: (i, 0)),
            pl.BlockSpec((RB, H), lambda i: (i, 0)),
        ],
        out_shape=[
            jax.ShapeDtypeStruct((N, R), F32),
            jax.ShapeDtypeStruct((N, H), F32),
            jax.ShapeDtypeStruct((N, H), F32),
        ],
    )(x, h0, sums, wihxT, wihaT, bih, whhT, bhh,
      w2sxT, w2shT, b2s, w2dxT, w2dhT)


def _k8_body(x_r, h1_r, sums_r, cnts_r, wx_r, wh_r, wa_r, b1_r, w2_r, b2_r,
             out_r):
    cnt = cnts_r[0, :, L:L + 1] + cnts_r[1, :, L:L + 1]
    agg = (sums_r[0] + sums_r[1])[:, :L] / jnp.maximum(cnt, 1.0)
    hh = jax.nn.relu(_dot(x_r[...], wx_r[...]) + _dot(h1_r[...], wh_r[...])
                     + _dot(agg, wa_r[...]) + b1_r[...])
    out_r[...] = _dot(hh, w2_r[...]) + b2_r[...]


def _k8(x, h1, sums, cnts, wxT, whT, waT, b1, w2T, b2):
    return pl.pallas_call(
        _k8_body,
        grid=(N // RB,),
        in_specs=[
            pl.BlockSpec((RB, F), lambda i: (i, 0)),
            pl.BlockSpec((RB, R), lambda i: (i, 0)),
            pl.BlockSpec((NC, RB, WS), lambda i: (0, i, 0)),
            pl.BlockSpec((NC, RB, WS), lambda i: (0, i, 0)),
            _full((F, H)), _full((R, H)), _full((L, H)), _full((1, H)),
            _full((H, O)), _full((1, O)),
        ],
        out_specs=pl.BlockSpec((RB, O), lambda i: (i, 0)),
        out_shape=jax.ShapeDtypeStruct((N, O), F32),
    )(x, h1, sums, cnts, wxT, whT, waT, b1, w2T, b2)


def kernel(x, edge_index, edge_attr, hidden, params):
    p = params
    row2d = edge_index[0].reshape(NW, CPW, C)
    col2d = edge_index[1].reshape(NW, CPW, C)
    h0 = hidden[0]

    zs = jnp.zeros((NP, WS), F32)

    w1 = p["e1_w1"]
    b1 = p["e1_b1"].reshape(1, H)
    a1, bb1 = _k0(x, w1[:, :F].T, w1[:, F:2 * F].T, b1)
    g1a, g1b = _sc_gather2(a1, bb1, row2d, col2d)
    ea1 = _k2(g1a, g1b, edge_attr, w1[:, 2 * F:].T, p["e1_w2"].T,
              p["e1_b2"].reshape(1, L))
    sums1 = _sc_scatter(ea1, edge_index[1], zs).reshape(NC, NP, WS)

    wih = p["w_ih"]
    w2 = p["e2_w1"]
    h1, a2, bb2 = _k4(
        x, h0, sums1,
        wih[:, :F].T, wih[:, F:].T, p["b_ih"].reshape(1, 3 * R),
        p["w_hh"].T, p["b_hh"].reshape(1, 3 * R),
        w2[:, :F].T, w2[:, F:256].T, p["e2_b1"].reshape(1, H),
        w2[:, 256:256 + F].T, w2[:, 256 + F:512].T,
    )
    g2a, g2b = _sc_gather2(a2, bb2, row2d, col2d)
    ea2 = _k6(g2a, g2b, edge_attr, ea1, w2[:, 512:512 + FE].T,
              w2[:, 512 + FE:].T, p["e2_w2"].T, p["e2_b2"].reshape(1, L))
    sums2 = _sc_scatter(ea2, edge_index[1], zs).reshape(NC, NP, WS)

    w3 = p["n2_w1"]
    out = _k8(x, h1, sums2, sums1,
              w3[:, :F].T, w3[:, F:256].T, w3[:, 256:].T,
              p["n2_b1"].reshape(1, H), p["n2_w2"].T,
              p["n2_b2"].reshape(1, O))
    return (out, h1[None])


# speedup vs baseline: 2.7094x; 1.3178x over previous
import functools

import jax
import jax.numpy as jnp
from jax import lax
from jax.experimental import pallas as pl
from jax.experimental.pallas import tpu as pltpu
from jax.experimental.pallas import tpu_sc as plsc

F32 = jnp.float32
BF16 = jnp.bfloat16

N = 10000
E = 320000
F = 128
FE = 16
H = 256
R = 128
L = 64
O = 128

NC = 2
NS = 16
NW = NC * NS
EPW = E // NW
C = 80
CPW = EPW // C
NP = 10240
NPT = NP // NS
WS = 128

_SC_MESH = plsc.VectorSubcoreMesh(core_axis_name="c", subcore_axis_name="s",
                                  num_cores=NC, num_subcores=NS)


def _dot(a, b):
    return jnp.dot(a, b, preferred_element_type=F32)


HP = H // 2


def _rne16(u):
    return u + jnp.uint32(0x7FFF) + ((u >> 16) & jnp.uint32(1))


def _pack_bf16(v):
    lo = _rne16(jax.lax.bitcast_convert_type(v[:, :HP], jnp.uint32))
    hi = _rne16(jax.lax.bitcast_convert_type(v[:, HP:], jnp.uint32))
    packed = (hi & jnp.uint32(0xFFFF0000)) | (lo >> 16)
    return jax.lax.bitcast_convert_type(packed, F32)


def _unpack_bf16(g):
    u = jax.lax.bitcast_convert_type(g, jnp.uint32)
    lo = jax.lax.bitcast_convert_type(u << 16, F32)
    hi = jax.lax.bitcast_convert_type(u & jnp.uint32(0xFFFF0000), F32)
    return jnp.concatenate([lo, hi], axis=1)


@functools.partial(
    pl.kernel,
    out_type=(
        jax.ShapeDtypeStruct((E, HP), F32),
        jax.ShapeDtypeStruct((E, HP), F32),
    ),
    mesh=_SC_MESH,
    scratch_types=[
        pltpu.VMEM((CPW, C), jnp.int32),
        pltpu.VMEM((CPW, C), jnp.int32),
        pltpu.VMEM((C, HP), F32),
        pltpu.VMEM((C, HP), F32),
    ],
)
def _sc_gather2(a_hbm, b_hbm, row_hbm, col_hbm, oa_hbm, ob_hbm,
                ridx, cidx, bufa, bufb):
    wid = lax.axis_index("s") * NC + lax.axis_index("c")
    pltpu.sync_copy(row_hbm.at[wid], ridx)
    pltpu.sync_copy(col_hbm.at[wid], cidx)

    def step(j, _):
        pltpu.sync_copy(a_hbm.at[ridx.at[j]], bufa)
        pltpu.sync_copy(b_hbm.at[cidx.at[j]], bufb)
        pltpu.sync_copy(bufa, oa_hbm.at[pl.ds(wid * EPW + j * C, C)])
        pltpu.sync_copy(bufb, ob_hbm.at[pl.ds(wid * EPW + j * C, C)])
        return 0

    lax.fori_loop(0, CPW, step, 0)


def _make_sc_scatter(width):
    assert width == WS

    @functools.partial(
        pl.kernel,
        out_type=jax.ShapeDtypeStruct((NC * NP, width), F32),
        mesh=_SC_MESH,
        scratch_types=[
            pltpu.VMEM((EPW,), jnp.int32),
            pltpu.VMEM((C, width), F32),
            pltpu.VMEM_SHARED((NP, width), F32),
        ],
    )
    def scatter(vals_hbm, colf_hbm, zs_hbm, sum_out, cidx, vals, sum_sh):
        cid = lax.axis_index("c")
        sid = lax.axis_index("s")
        wid = sid * NC + cid
        pltpu.sync_copy(zs_hbm.at[pl.ds(0, C)], vals)

        def zstep(t, _):
            pltpu.sync_copy(vals, sum_sh.at[pl.ds(sid * NPT + t * C, C)])
            return 0

        lax.fori_loop(0, NPT // C, zstep, 0)
        pltpu.sync_copy(colf_hbm.at[pl.ds(wid * EPW, EPW)], cidx)
        plsc.subcore_barrier()

        def step(j, _):
            pltpu.sync_copy(vals_hbm.at[pl.ds(wid * EPW + j * C, C)], vals)

            def sub(k, _2):
                idx16 = cidx[pl.ds(j * C + k * 16, 16)]
                pltpu.sync_copy(vals.at[pl.ds(k * 16, 16)],
                                sum_sh.at[idx16], add=True)
                return 0

            lax.fori_loop(0, C // 16, sub, 0)
            return 0

        lax.fori_loop(0, CPW, step, 0)
        plsc.subcore_barrier()

        def wstep(t, _):
            pltpu.sync_copy(sum_sh.at[pl.ds(sid * NPT + t * C, C)], vals)
            pltpu.sync_copy(vals,
                            sum_out.at[pl.ds(cid * NP + sid * NPT + t * C, C)])
            return 0

        lax.fori_loop(0, NPT // C, wstep, 0)

    return scatter


_sc_scatter = _make_sc_scatter(WS)


RB = 2000
EB = 4000


def _full(shape):
    return pl.BlockSpec(shape, lambda i: (0,) * len(shape))


def _k0_body(x_r, wsx_r, wdx_r, b1_r, a_r, b_r):
    x = x_r[...]
    a_r[...] = _pack_bf16(_dot(x, wsx_r[...]) + b1_r[...])
    b_r[...] = _pack_bf16(_dot(x, wdx_r[...]))


def _k0(x, wsxT, wdxT, b1):
    return pl.pallas_call(
        _k0_body,
        grid=(N // RB,),
        in_specs=[
            pl.BlockSpec((RB, F), lambda i: (i, 0)),
            _full((F, H)), _full((F, H)), _full((1, H)),
        ],
        out_specs=[pl.BlockSpec((RB, HP), lambda i: (i, 0))] * 2,
        out_shape=[jax.ShapeDtypeStruct((N, HP), F32)] * 2,
    )(x, wsxT, wdxT, b1)


def _k2_body(ga_r, gb_r, ea_r, weT_r, w2T_r, b2_r, out_r):
    h = jax.nn.relu(_unpack_bf16(ga_r[...]) + _unpack_bf16(gb_r[...])
                    + _dot(ea_r[...], weT_r[...]))
    msg = _dot(h, w2T_r[...]) + b2_r[...]
    out_r[...] = jnp.concatenate(
        [msg, jnp.ones((EB, FE), F32), jnp.zeros((EB, WS - L - FE), F32)],
        axis=1)


def _k2(ga, gb, ea, weT, w2T, b2):
    return pl.pallas_call(
        _k2_body,
        grid=(E // EB,),
        in_specs=[
            pl.BlockSpec((EB, HP), lambda i: (i, 0)),
            pl.BlockSpec((EB, HP), lambda i: (i, 0)),
            pl.BlockSpec((EB, FE), lambda i: (i, 0)),
            _full((FE, H)), _full((H, L)), _full((1, L)),
        ],
        out_specs=pl.BlockSpec((EB, WS), lambda i: (i, 0)),
        out_shape=jax.ShapeDtypeStruct((E, WS), F32),
    )(ga, gb, ea, weT, w2T, b2)


def _k6_body(ga_r, gb_r, ea_r, ea1_r, weaT_r, wea1T_r, w2T_r, b2_r, out_r):
    ea1 = ea1_r[...][:, :L]
    h = jax.nn.relu(_unpack_bf16(ga_r[...]) + _unpack_bf16(gb_r[...])
                    + _dot(ea_r[...], weaT_r[...])
                    + _dot(ea1, wea1T_r[...]))
    msg = _dot(h, w2T_r[...]) + b2_r[...]
    out_r[...] = jnp.concatenate([msg, jnp.zeros((EB, WS - L), F32)], axis=1)


def _k6(ga, gb, ea, ea1, weaT, wea1T, w2T, b2):
    return pl.pallas_call(
        _k6_body,
        grid=(E // EB,),
        in_specs=[
            pl.BlockSpec((EB, HP), lambda i: (i, 0)),
            pl.BlockSpec((EB, HP), lambda i: (i, 0)),
            pl.BlockSpec((EB, FE), lambda i: (i, 0)),
            pl.BlockSpec((EB, WS), lambda i: (i, 0)),
            _full((FE, H)), _full((L, H)), _full((H, L)), _full((1, L)),
        ],
        out_specs=pl.BlockSpec((EB, WS), lambda i: (i, 0)),
        out_shape=jax.ShapeDtypeStruct((E, WS), F32),
    )(ga, gb, ea, ea1, weaT, wea1T, w2T, b2)


def _k4_body(x_r, h0_r, sums_r, wihx_r, wiha_r, bih_r, whh_r, bhh_r,
             w2sx_r, w2sh_r, b2s_r, w2dx_r, w2dh_r,
             h1_r, a2_r, b2_r):
    s = sums_r[0] + sums_r[1]
    agg = s[:, :L] / jnp.maximum(s[:, L:L + 1], 1.0)
    x = x_r[...]
    h0 = h0_r[...]
    gi = _dot(x, wihx_r[...]) + _dot(agg, wiha_r[...]) + bih_r[...]
    gh = _dot(h0, whh_r[...]) + bhh_r[...]
    r = jax.nn.sigmoid(gi[:, :R] + gh[:, :R])
    z = jax.nn.sigmoid(gi[:, R:2 * R] + gh[:, R:2 * R])
    n = jnp.tanh(gi[:, 2 * R:] + r * gh[:, 2 * R:])
    h1 = (1.0 - z) * n + z * h0
    h1_r[...] = h1
    a2_r[...] = _pack_bf16(_dot(x, w2sx_r[...]) + _dot(h1, w2sh_r[...])
                           + b2s_r[...])
    b2_r[...] = _pack_bf16(_dot(x, w2dx_r[...]) + _dot(h1, w2dh_r[...]))


def _k4(x, h0, sums, wihxT, wihaT, bih, whhT, bhh,
        w2sxT, w2shT, b2s, w2dxT, w2dhT):
    return pl.pallas_call(
        _k4_body,
        grid=(N // RB,),
        in_specs=[
            pl.BlockSpec((RB, F), lambda i: (i, 0)),
            pl.BlockSpec((RB, R), lambda i: (i, 0)),
            pl.BlockSpec((NC, RB, WS), lambda i: (0, i, 0)),
            _full((F, 3 * R)), _full((L, 3 * R)), _full((1, 3 * R)),
            _full((R, 3 * R)), _full((1, 3 * R)),
            _full((F, H)), _full((R, H)), _full((1, H)),
            _full((F, H)), _full((R, H)),
        ],
        out_specs=[
            pl.BlockSpec((RB, R), lambda i: (i, 0)),
            pl.BlockSpec((RB, HP), lambda i: (i, 0)),
            pl.BlockSpec((RB, HP), lambda i: (i, 0)),
        ],
        out_shape=[
            jax.ShapeDtypeStruct((N, R), F32),
            jax.ShapeDtypeStruct((N, HP), F32),
            jax.ShapeDtypeStruct((N, HP), F32),
        ],
    )(x, h0, sums, wihxT, wihaT, bih, whhT, bhh,
      w2sxT, w2shT, b2s, w2dxT, w2dhT)


def _k8_body(x_r, h1_r, sums_r, cnts_r, wx_r, wh_r, wa_r, b1_r, w2_r, b2_r,
             out_r):
    cnt = cnts_r[0, :, L:L + 1] + cnts_r[1, :, L:L + 1]
    agg = (sums_r[0] + sums_r[1])[:, :L] / jnp.maximum(cnt, 1.0)
    hh = jax.nn.relu(_dot(x_r[...], wx_r[...]) + _dot(h1_r[...], wh_r[...])
                     + _dot(agg, wa_r[...]) + b1_r[...])
    out_r[...] = _dot(hh, w2_r[...]) + b2_r[...]


def _k8(x, h1, sums, cnts, wxT, whT, waT, b1, w2T, b2):
    return pl.pallas_call(
        _k8_body,
        grid=(N // RB,),
        in_specs=[
            pl.BlockSpec((RB, F), lambda i: (i, 0)),
            pl.BlockSpec((RB, R), lambda i: (i, 0)),
            pl.BlockSpec((NC, RB, WS), lambda i: (0, i, 0)),
            pl.BlockSpec((NC, RB, WS), lambda i: (0, i, 0)),
            _full((F, H)), _full((R, H)), _full((L, H)), _full((1, H)),
            _full((H, O)), _full((1, O)),
        ],
        out_specs=pl.BlockSpec((RB, O), lambda i: (i, 0)),
        out_shape=jax.ShapeDtypeStruct((N, O), F32),
    )(x, h1, sums, cnts, wxT, whT, waT, b1, w2T, b2)


def kernel(x, edge_index, edge_attr, hidden, params):
    p = params
    row2d = edge_index[0].reshape(NW, CPW, C)
    col2d = edge_index[1].reshape(NW, CPW, C)
    h0 = hidden[0]

    zs = jnp.zeros((NP, WS), F32)

    w1 = p["e1_w1"]
    b1 = p["e1_b1"].reshape(1, H)
    a1, bb1 = _k0(x, w1[:, :F].T, w1[:, F:2 * F].T, b1)
    g1a, g1b = _sc_gather2(a1, bb1, row2d, col2d)
    ea1 = _k2(g1a, g1b, edge_attr, w1[:, 2 * F:].T, p["e1_w2"].T,
              p["e1_b2"].reshape(1, L))
    sums1 = _sc_scatter(ea1, edge_index[1], zs).reshape(NC, NP, WS)

    wih = p["w_ih"]
    w2 = p["e2_w1"]
    h1, a2, bb2 = _k4(
        x, h0, sums1,
        wih[:, :F].T, wih[:, F:].T, p["b_ih"].reshape(1, 3 * R),
        p["w_hh"].T, p["b_hh"].reshape(1, 3 * R),
        w2[:, :F].T, w2[:, F:256].T, p["e2_b1"].reshape(1, H),
        w2[:, 256:256 + F].T, w2[:, 256 + F:512].T,
    )
    g2a, g2b = _sc_gather2(a2, bb2, row2d, col2d)
    ea2 = _k6(g2a, g2b, edge_attr, ea1, w2[:, 512:512 + FE].T,
              w2[:, 512 + FE:].T, p["e2_w2"].T, p["e2_b2"].reshape(1, L))
    sums2 = _sc_scatter(ea2, edge_index[1], zs).reshape(NC, NP, WS)

    w3 = p["n2_w1"]
    out = _k8(x, h1, sums2, sums1,
              w3[:, :F].T, w3[:, F:256].T, w3[:, 256:].T,
              p["n2_b1"].reshape(1, H), p["n2_w2"].T,
              p["n2_b2"].reshape(1, O))
    return (out, h1[None])


# speedup vs baseline: 3.2766x; 1.2093x over previous
import functools

import jax
import jax.numpy as jnp
from jax import lax
from jax.experimental import pallas as pl
from jax.experimental.pallas import tpu as pltpu
from jax.experimental.pallas import tpu_sc as plsc

F32 = jnp.float32
BF16 = jnp.bfloat16

N = 10000
E = 320000
F = 128
FE = 16
H = 256
R = 128
L = 64
O = 128

NC = 2
NS = 16
NW = NC * NS
EPW = E // NW
C = 80
CPW = EPW // C
NP = 10240
NPT = NP // NS
WS = 128

_SC_MESH = plsc.VectorSubcoreMesh(core_axis_name="c", subcore_axis_name="s",
                                  num_cores=NC, num_subcores=NS)


def _dot(a, b):
    return jnp.dot(a, b, preferred_element_type=F32)


HP = H // 2


def _rne16(u):
    return u + jnp.uint32(0x7FFF) + ((u >> 16) & jnp.uint32(1))


def _pack_bf16(v):
    lo = _rne16(jax.lax.bitcast_convert_type(v[:, :HP], jnp.uint32))
    hi = _rne16(jax.lax.bitcast_convert_type(v[:, HP:], jnp.uint32))
    packed = (hi & jnp.uint32(0xFFFF0000)) | (lo >> 16)
    return jax.lax.bitcast_convert_type(packed, F32)


def _unpack_bf16(g):
    u = jax.lax.bitcast_convert_type(g, jnp.uint32)
    lo = jax.lax.bitcast_convert_type(u << 16, F32)
    hi = jax.lax.bitcast_convert_type(u & jnp.uint32(0xFFFF0000), F32)
    return jnp.concatenate([lo, hi], axis=1)


@functools.partial(
    pl.kernel,
    out_type=(
        jax.ShapeDtypeStruct((E, HP), F32),
        jax.ShapeDtypeStruct((E, HP), F32),
    ),
    mesh=_SC_MESH,
    scratch_types=[
        pltpu.VMEM((CPW, C), jnp.int32),
        pltpu.VMEM((CPW, C), jnp.int32),
        pltpu.VMEM((C, HP), F32),
        pltpu.VMEM((C, HP), F32),
        pltpu.VMEM((C, HP), F32),
        pltpu.VMEM((C, HP), F32),
        pltpu.SemaphoreType.DMA,
        pltpu.SemaphoreType.DMA,
        pltpu.SemaphoreType.DMA,
        pltpu.SemaphoreType.DMA,
        pltpu.SemaphoreType.DMA,
        pltpu.SemaphoreType.DMA,
        pltpu.SemaphoreType.DMA,
        pltpu.SemaphoreType.DMA,
    ],
)
def _sc_gather2(a_hbm, b_hbm, row_hbm, col_hbm, oa_hbm, ob_hbm,
                ridx, cidx, bufa0, bufb0, bufa1, bufb1,
                ga0, gb0, wa0, wb0, ga1, gb1, wa1, wb1):
    wid = lax.axis_index("s") * NC + lax.axis_index("c")
    pltpu.sync_copy(row_hbm.at[wid], ridx)
    pltpu.sync_copy(col_hbm.at[wid], cidx)
    bufa = (bufa0, bufa1)
    bufb = (bufb0, bufb1)
    ga = (ga0, ga1)
    gb = (gb0, gb1)
    wa = (wa0, wa1)
    wb = (wb0, wb1)

    def out_at(j):
        return pl.ds(wid * EPW + j * C, C)

    def step(i, _):
        for b in (0, 1):
            j = 2 * i + b
            @pl.when(i > 0)
            def _():
                pltpu.make_async_copy(bufa[b], oa_hbm.at[out_at(j)],
                                      wa[b]).wait()
                pltpu.make_async_copy(bufb[b], ob_hbm.at[out_at(j)],
                                      wb[b]).wait()
            pltpu.async_copy(a_hbm.at[ridx.at[j]], bufa[b], ga[b])
            pltpu.async_copy(b_hbm.at[cidx.at[j]], bufb[b], gb[b])
        for b in (0, 1):
            j = 2 * i + b
            pltpu.make_async_copy(a_hbm.at[ridx.at[j]], bufa[b], ga[b]).wait()
            pltpu.make_async_copy(b_hbm.at[cidx.at[j]], bufb[b], gb[b]).wait()
            pltpu.async_copy(bufa[b], oa_hbm.at[out_at(j)], wa[b])
            pltpu.async_copy(bufb[b], ob_hbm.at[out_at(j)], wb[b])
        return 0

    lax.fori_loop(0, CPW // 2, step, 0)
    for b in (0, 1):
        pltpu.make_async_copy(bufa[b], oa_hbm.at[out_at(0)], wa[b]).wait()
        pltpu.make_async_copy(bufb[b], ob_hbm.at[out_at(0)], wb[b]).wait()
    if CPW % 2:
        j = CPW - 1
        pltpu.sync_copy(a_hbm.at[ridx.at[j]], bufa0)
        pltpu.sync_copy(b_hbm.at[cidx.at[j]], bufb0)
        pltpu.sync_copy(bufa0, oa_hbm.at[out_at(j)])
        pltpu.sync_copy(bufb0, ob_hbm.at[out_at(j)])


def _make_sc_scatter(width):
    assert width == WS

    @functools.partial(
        pl.kernel,
        out_type=jax.ShapeDtypeStruct((NC * NP, width), F32),
        mesh=_SC_MESH,
        scratch_types=[
            pltpu.VMEM((EPW,), jnp.int32),
            pltpu.VMEM((C, width), F32),
            pltpu.VMEM_SHARED((NP, width), F32),
        ],
    )
    def scatter(vals_hbm, colf_hbm, zs_hbm, sum_out, cidx, vals, sum_sh):
        cid = lax.axis_index("c")
        sid = lax.axis_index("s")
        wid = sid * NC + cid
        pltpu.sync_copy(zs_hbm.at[pl.ds(0, C)], vals)

        def zstep(t, _):
            pltpu.sync_copy(vals, sum_sh.at[pl.ds(sid * NPT + t * C, C)])
            return 0

        lax.fori_loop(0, NPT // C, zstep, 0)
        pltpu.sync_copy(colf_hbm.at[pl.ds(wid * EPW, EPW)], cidx)
        plsc.subcore_barrier()

        def step(j, _):
            pltpu.sync_copy(vals_hbm.at[pl.ds(wid * EPW + j * C, C)], vals)

            def sub(k, _2):
                idx16 = cidx[pl.ds(j * C + k * 16, 16)]
                pltpu.sync_copy(vals.at[pl.ds(k * 16, 16)],
                                sum_sh.at[idx16], add=True)
                return 0

            lax.fori_loop(0, C // 16, sub, 0)
            return 0

        lax.fori_loop(0, CPW, step, 0)
        plsc.subcore_barrier()

        def wstep(t, _):
            pltpu.sync_copy(sum_sh.at[pl.ds(sid * NPT + t * C, C)], vals)
            pltpu.sync_copy(vals,
                            sum_out.at[pl.ds(cid * NP + sid * NPT + t * C, C)])
            return 0

        lax.fori_loop(0, NPT // C, wstep, 0)

    return scatter


_sc_scatter = _make_sc_scatter(WS)


RB = 2000
EB = 4000


def _full(shape):
    return pl.BlockSpec(shape, lambda i: (0,) * len(shape))


def _k0_body(x_r, wsx_r, wdx_r, b1_r, a_r, b_r):
    x = x_r[...]
    a_r[...] = _pack_bf16(_dot(x, wsx_r[...]) + b1_r[...])
    b_r[...] = _pack_bf16(_dot(x, wdx_r[...]))


def _k0(x, wsxT, wdxT, b1):
    return pl.pallas_call(
        _k0_body,
        grid=(N // RB,),
        in_specs=[
            pl.BlockSpec((RB, F), lambda i: (i, 0)),
            _full((F, H)), _full((F, H)), _full((1, H)),
        ],
        out_specs=[pl.BlockSpec((RB, HP), lambda i: (i, 0))] * 2,
        out_shape=[jax.ShapeDtypeStruct((N, HP), F32)] * 2,
    )(x, wsxT, wdxT, b1)


def _k2_body(ga_r, gb_r, ea_r, weT_r, w2T_r, b2_r, out_r):
    h = jax.nn.relu(_unpack_bf16(ga_r[...]) + _unpack_bf16(gb_r[...])
                    + _dot(ea_r[...], weT_r[...]))
    msg = _dot(h, w2T_r[...]) + b2_r[...]
    out_r[...] = jnp.concatenate(
        [msg, jnp.ones((EB, FE), F32), jnp.zeros((EB, WS - L - FE), F32)],
        axis=1)


def _k2(ga, gb, ea, weT, w2T, b2):
    return pl.pallas_call(
        _k2_body,
        grid=(E // EB,),
        in_specs=[
            pl.BlockSpec((EB, HP), lambda i: (i, 0)),
            pl.BlockSpec((EB, HP), lambda i: (i, 0)),
            pl.BlockSpec((EB, FE), lambda i: (i, 0)),
            _full((FE, H)), _full((H, L)), _full((1, L)),
        ],
        out_specs=pl.BlockSpec((EB, WS), lambda i: (i, 0)),
        out_shape=jax.ShapeDtypeStruct((E, WS), F32),
    )(ga, gb, ea, weT, w2T, b2)


def _k6_body(ga_r, gb_r, ea_r, ea1_r, weaT_r, wea1T_r, w2T_r, b2_r, out_r):
    ea1 = ea1_r[...][:, :L]
    h = jax.nn.relu(_unpack_bf16(ga_r[...]) + _unpack_bf16(gb_r[...])
                    + _dot(ea_r[...], weaT_r[...])
                    + _dot(ea1, wea1T_r[...]))
    msg = _dot(h, w2T_r[...]) + b2_r[...]
    out_r[...] = jnp.concatenate([msg, jnp.zeros((EB, WS - L), F32)], axis=1)


def _k6(ga, gb, ea, ea1, weaT, wea1T, w2T, b2):
    return pl.pallas_call(
        _k6_body,
        grid=(E // EB,),
        in_specs=[
            pl.BlockSpec((EB, HP), lambda i: (i, 0)),
            pl.BlockSpec((EB, HP), lambda i: (i, 0)),
            pl.BlockSpec((EB, FE), lambda i: (i, 0)),
            pl.BlockSpec((EB, WS), lambda i: (i, 0)),
            _full((FE, H)), _full((L, H)), _full((H, L)), _full((1, L)),
        ],
        out_specs=pl.BlockSpec((EB, WS), lambda i: (i, 0)),
        out_shape=jax.ShapeDtypeStruct((E, WS), F32),
    )(ga, gb, ea, ea1, weaT, wea1T, w2T, b2)


def _k4_body(x_r, h0_r, sums_r, wihx_r, wiha_r, bih_r, whh_r, bhh_r,
             w2sx_r, w2sh_r, b2s_r, w2dx_r, w2dh_r,
             h1_r, a2_r, b2_r):
    s = sums_r[0] + sums_r[1]
    agg = s[:, :L] / jnp.maximum(s[:, L:L + 1], 1.0)
    x = x_r[...]
    h0 = h0_r[...]
    gi = _dot(x, wihx_r[...]) + _dot(agg, wiha_r[...]) + bih_r[...]
    gh = _dot(h0, whh_r[...]) + bhh_r[...]
    r = jax.nn.sigmoid(gi[:, :R] + gh[:, :R])
    z = jax.nn.sigmoid(gi[:, R:2 * R] + gh[:, R:2 * R])
    n = jnp.tanh(gi[:, 2 * R:] + r * gh[:, 2 * R:])
    h1 = (1.0 - z) * n + z * h0
    h1_r[...] = h1
    a2_r[...] = _pack_bf16(_dot(x, w2sx_r[...]) + _dot(h1, w2sh_r[...])
                           + b2s_r[...])
    b2_r[...] = _pack_bf16(_dot(x, w2dx_r[...]) + _dot(h1, w2dh_r[...]))


def _k4(x, h0, sums, wihxT, wihaT, bih, whhT, bhh,
        w2sxT, w2shT, b2s, w2dxT, w2dhT):
    return pl.pallas_call(
        _k4_body,
        grid=(N // RB,),
        in_specs=[
            pl.BlockSpec((RB, F), lambda i: (i, 0)),
            pl.BlockSpec((RB, R), lambda i: (i, 0)),
            pl.BlockSpec((NC, RB, WS), lambda i: (0, i, 0)),
            _full((F, 3 * R)), _full((L, 3 * R)), _full((1, 3 * R)),
            _full((R, 3 * R)), _full((1, 3 * R)),
            _full((F, H)), _full((R, H)), _full((1, H)),
            _full((F, H)), _full((R, H)),
        ],
        out_specs=[
            pl.BlockSpec((RB, R), lambda i: (i, 0)),
            pl.BlockSpec((RB, HP), lambda i: (i, 0)),
            pl.BlockSpec((RB, HP), lambda i: (i, 0)),
        ],
        out_shape=[
            jax.ShapeDtypeStruct((N, R), F32),
            jax.ShapeDtypeStruct((N, HP), F32),
            jax.ShapeDtypeStruct((N, HP), F32),
        ],
    )(x, h0, sums, wihxT, wihaT, bih, whhT, bhh,
      w2sxT, w2shT, b2s, w2dxT, w2dhT)


def _k8_body(x_r, h1_r, sums_r, cnts_r, wx_r, wh_r, wa_r, b1_r, w2_r, b2_r,
             out_r):
    cnt = cnts_r[0, :, L:L + 1] + cnts_r[1, :, L:L + 1]
    agg = (sums_r[0] + sums_r[1])[:, :L] / jnp.maximum(cnt, 1.0)
    hh = jax.nn.relu(_dot(x_r[...], wx_r[...]) + _dot(h1_r[...], wh_r[...])
                     + _dot(agg, wa_r[...]) + b1_r[...])
    out_r[...] = _dot(hh, w2_r[...]) + b2_r[...]


def _k8(x, h1, sums, cnts, wxT, whT, waT, b1, w2T, b2):
    return pl.pallas_call(
        _k8_body,
        grid=(N // RB,),
        in_specs=[
            pl.BlockSpec((RB, F), lambda i: (i, 0)),
            pl.BlockSpec((RB, R), lambda i: (i, 0)),
            pl.BlockSpec((NC, RB, WS), lambda i: (0, i, 0)),
            pl.BlockSpec((NC, RB, WS), lambda i: (0, i, 0)),
            _full((F, H)), _full((R, H)), _full((L, H)), _full((1, H)),
            _full((H, O)), _full((1, O)),
        ],
        out_specs=pl.BlockSpec((RB, O), lambda i: (i, 0)),
        out_shape=jax.ShapeDtypeStruct((N, O), F32),
    )(x, h1, sums, cnts, wxT, whT, waT, b1, w2T, b2)


def kernel(x, edge_index, edge_attr, hidden, params):
    p = params
    row2d = edge_index[0].reshape(NW, CPW, C)
    col2d = edge_index[1].reshape(NW, CPW, C)
    h0 = hidden[0]

    zs = jnp.zeros((NP, WS), F32)

    w1 = p["e1_w1"]
    b1 = p["e1_b1"].reshape(1, H)
    a1, bb1 = _k0(x, w1[:, :F].T, w1[:, F:2 * F].T, b1)
    g1a, g1b = _sc_gather2(a1, bb1, row2d, col2d)
    ea1 = _k2(g1a, g1b, edge_attr, w1[:, 2 * F:].T, p["e1_w2"].T,
              p["e1_b2"].reshape(1, L))
    sums1 = _sc_scatter(ea1, edge_index[1], zs).reshape(NC, NP, WS)

    wih = p["w_ih"]
    w2 = p["e2_w1"]
    h1, a2, bb2 = _k4(
        x, h0, sums1,
        wih[:, :F].T, wih[:, F:].T, p["b_ih"].reshape(1, 3 * R),
        p["w_hh"].T, p["b_hh"].reshape(1, 3 * R),
        w2[:, :F].T, w2[:, F:256].T, p["e2_b1"].reshape(1, H),
        w2[:, 256:256 + F].T, w2[:, 256 + F:512].T,
    )
    g2a, g2b = _sc_gather2(a2, bb2, row2d, col2d)
    ea2 = _k6(g2a, g2b, edge_attr, ea1, w2[:, 512:512 + FE].T,
              w2[:, 512 + FE:].T, p["e2_w2"].T, p["e2_b2"].reshape(1, L))
    sums2 = _sc_scatter(ea2, edge_index[1], zs).reshape(NC, NP, WS)

    w3 = p["n2_w1"]
    out = _k8(x, h1, sums2, sums1,
              w3[:, :F].T, w3[:, F:256].T, w3[:, 256:].T,
              p["n2_b1"].reshape(1, H), p["n2_w2"].T,
              p["n2_b2"].reshape(1, O))
    return (out, h1[None])


# speedup vs baseline: 3.6282x; 1.1073x over previous
import functools

import jax
import jax.numpy as jnp
from jax import lax
from jax.experimental import pallas as pl
from jax.experimental.pallas import tpu as pltpu
from jax.experimental.pallas import tpu_sc as plsc

F32 = jnp.float32
BF16 = jnp.bfloat16

N = 10000
E = 320000
F = 128
FE = 16
H = 256
R = 128
L = 64
O = 128

NC = 2
NS = 16
NW = NC * NS
EPW = E // NW
C = 80
CPW = EPW // C
NP = 10240
NPT = NP // NS
WS = 128

_SC_MESH = plsc.VectorSubcoreMesh(core_axis_name="c", subcore_axis_name="s",
                                  num_cores=NC, num_subcores=NS)


def _dot(a, b):
    return jnp.dot(a, b, preferred_element_type=F32)


HP = H // 2


def _rne16(u):
    return u + jnp.uint32(0x7FFF) + ((u >> 16) & jnp.uint32(1))


def _pack_bf16(v):
    lo = _rne16(jax.lax.bitcast_convert_type(v[:, :HP], jnp.uint32))
    hi = _rne16(jax.lax.bitcast_convert_type(v[:, HP:], jnp.uint32))
    packed = (hi & jnp.uint32(0xFFFF0000)) | (lo >> 16)
    return jax.lax.bitcast_convert_type(packed, F32)


def _unpack_bf16(g):
    u = jax.lax.bitcast_convert_type(g, jnp.uint32)
    lo = jax.lax.bitcast_convert_type(u << 16, F32)
    hi = jax.lax.bitcast_convert_type(u & jnp.uint32(0xFFFF0000), F32)
    return jnp.concatenate([lo, hi], axis=1)


@functools.partial(
    pl.kernel,
    out_type=(
        jax.ShapeDtypeStruct((E, HP), F32),
        jax.ShapeDtypeStruct((E, HP), F32),
    ),
    mesh=_SC_MESH,
    scratch_types=[
        pltpu.VMEM((CPW, C), jnp.int32),
        pltpu.VMEM((CPW, C), jnp.int32),
        pltpu.VMEM((C, HP), F32),
        pltpu.VMEM((C, HP), F32),
        pltpu.VMEM((C, HP), F32),
        pltpu.VMEM((C, HP), F32),
        pltpu.SemaphoreType.DMA,
        pltpu.SemaphoreType.DMA,
        pltpu.SemaphoreType.DMA,
        pltpu.SemaphoreType.DMA,
        pltpu.SemaphoreType.DMA,
        pltpu.SemaphoreType.DMA,
        pltpu.SemaphoreType.DMA,
        pltpu.SemaphoreType.DMA,
    ],
)
def _sc_gather2(a_hbm, b_hbm, row_hbm, col_hbm, oa_hbm, ob_hbm,
                ridx, cidx, bufa0, bufb0, bufa1, bufb1,
                ga0, gb0, wa0, wb0, ga1, gb1, wa1, wb1):
    wid = lax.axis_index("s") * NC + lax.axis_index("c")
    pltpu.sync_copy(row_hbm.at[wid], ridx)
    pltpu.sync_copy(col_hbm.at[wid], cidx)
    bufa = (bufa0, bufa1)
    bufb = (bufb0, bufb1)
    ga = (ga0, ga1)
    gb = (gb0, gb1)
    wa = (wa0, wa1)
    wb = (wb0, wb1)

    def out_at(j):
        return pl.ds(wid * EPW + j * C, C)

    def step(i, _):
        for b in (0, 1):
            j = 2 * i + b
            @pl.when(i > 0)
            def _():
                pltpu.make_async_copy(bufa[b], oa_hbm.at[out_at(j)],
                                      wa[b]).wait()
                pltpu.make_async_copy(bufb[b], ob_hbm.at[out_at(j)],
                                      wb[b]).wait()
            pltpu.async_copy(a_hbm.at[ridx.at[j]], bufa[b], ga[b])
            pltpu.async_copy(b_hbm.at[cidx.at[j]], bufb[b], gb[b])
        for b in (0, 1):
            j = 2 * i + b
            pltpu.make_async_copy(a_hbm.at[ridx.at[j]], bufa[b], ga[b]).wait()
            pltpu.make_async_copy(b_hbm.at[cidx.at[j]], bufb[b], gb[b]).wait()
            pltpu.async_copy(bufa[b], oa_hbm.at[out_at(j)], wa[b])
            pltpu.async_copy(bufb[b], ob_hbm.at[out_at(j)], wb[b])
        return 0

    lax.fori_loop(0, CPW // 2, step, 0)
    for b in (0, 1):
        pltpu.make_async_copy(bufa[b], oa_hbm.at[out_at(0)], wa[b]).wait()
        pltpu.make_async_copy(bufb[b], ob_hbm.at[out_at(0)], wb[b]).wait()
    if CPW % 2:
        j = CPW - 1
        pltpu.sync_copy(a_hbm.at[ridx.at[j]], bufa0)
        pltpu.sync_copy(b_hbm.at[cidx.at[j]], bufb0)
        pltpu.sync_copy(bufa0, oa_hbm.at[out_at(j)])
        pltpu.sync_copy(bufb0, ob_hbm.at[out_at(j)])


def _make_sc_scatter(width):
    assert width == WS

    @functools.partial(
        pl.kernel,
        out_type=jax.ShapeDtypeStruct((NC * NP, width), F32),
        mesh=_SC_MESH,
        scratch_types=[
            pltpu.VMEM((EPW,), jnp.int32),
            pltpu.VMEM((C, width), F32),
            pltpu.VMEM((C, width), F32),
            pltpu.SemaphoreType.DMA,
            pltpu.SemaphoreType.DMA,
            pltpu.SemaphoreType.DMA,
            pltpu.SemaphoreType.DMA,
            pltpu.VMEM_SHARED((NP, width), F32),
        ],
    )
    def scatter(vals_hbm, colf_hbm, zs_hbm, sum_out, cidx, vals0, vals1,
                lv0, lv1, s0, s1, sum_sh):
        cid = lax.axis_index("c")
        sid = lax.axis_index("s")
        wid = sid * NC + cid
        vals = (vals0, vals1)
        lv = (lv0, lv1)
        sc = (s0, s1)
        pltpu.sync_copy(zs_hbm.at[pl.ds(0, C)], vals0)

        def zstep(t, _):
            pltpu.sync_copy(vals0, sum_sh.at[pl.ds(sid * NPT + t * C, C)])
            return 0

        lax.fori_loop(0, NPT // C, zstep, 0)
        pltpu.sync_copy(colf_hbm.at[pl.ds(wid * EPW, EPW)], cidx)
        plsc.subcore_barrier()

        def vals_at(j):
            return pl.ds(wid * EPW + j * C, C)

        def drain(b):
            for k in range(C // 16):
                pltpu.make_async_copy(vals[b].at[pl.ds(k * 16, 16)],
                                      sum_sh.at[pl.ds(0, 16)], sc[b]).wait()

        def fire(b, j):
            for k in range(C // 16):
                idx16 = cidx[pl.ds(j * C + k * 16, 16)]
                pltpu.async_copy(vals[b].at[pl.ds(k * 16, 16)],
                                 sum_sh.at[idx16], sc[b], add=True)

        def step(i, _):
            for b in (0, 1):
                @pl.when(i > 0)
                def _():
                    drain(b)
                pltpu.async_copy(vals_hbm.at[vals_at(2 * i + b)], vals[b],
                                 lv[b])
            for b in (0, 1):
                pltpu.make_async_copy(vals_hbm.at[vals_at(2 * i + b)],
                                      vals[b], lv[b]).wait()
                fire(b, 2 * i + b)
            return 0

        lax.fori_loop(0, CPW // 2, step, 0)
        for b in (0, 1):
            drain(b)
        if CPW % 2:
            j = CPW - 1
            pltpu.sync_copy(vals_hbm.at[vals_at(j)], vals0)
            fire(0, j)
            drain(0)
        plsc.subcore_barrier()

        def wstep(t, _):
            pltpu.sync_copy(sum_sh.at[pl.ds(sid * NPT + t * C, C)], vals0)
            pltpu.sync_copy(vals0,
                            sum_out.at[pl.ds(cid * NP + sid * NPT + t * C, C)])
            return 0

        lax.fori_loop(0, NPT // C, wstep, 0)

    return scatter


_sc_scatter = _make_sc_scatter(WS)


RB = 2000
EB = 4000


def _full(shape):
    return pl.BlockSpec(shape, lambda i: (0,) * len(shape))


def _k0_body(x_r, wsx_r, wdx_r, b1_r, a_r, b_r):
    x = x_r[...]
    a_r[...] = _pack_bf16(_dot(x, wsx_r[...]) + b1_r[...])
    b_r[...] = _pack_bf16(_dot(x, wdx_r[...]))


def _k0(x, wsxT, wdxT, b1):
    return pl.pallas_call(
        _k0_body,
        grid=(N // RB,),
        in_specs=[
            pl.BlockSpec((RB, F), lambda i: (i, 0)),
            _full((F, H)), _full((F, H)), _full((1, H)),
        ],
        out_specs=[pl.BlockSpec((RB, HP), lambda i: (i, 0))] * 2,
        out_shape=[jax.ShapeDtypeStruct((N, HP), F32)] * 2,
    )(x, wsxT, wdxT, b1)


def _k2_body(ga_r, gb_r, ea_r, weT_r, w2T_r, b2_r, out_r):
    h = jax.nn.relu(_unpack_bf16(ga_r[...]) + _unpack_bf16(gb_r[...])
                    + _dot(ea_r[...], weT_r[...]))
    msg = _dot(h, w2T_r[...]) + b2_r[...]
    out_r[...] = jnp.concatenate(
        [msg, jnp.ones((EB, FE), F32), jnp.zeros((EB, WS - L - FE), F32)],
        axis=1)


def _k2(ga, gb, ea, weT, w2T, b2):
    return pl.pallas_call(
        _k2_body,
        grid=(E // EB,),
        in_specs=[
            pl.BlockSpec((EB, HP), lambda i: (i, 0)),
            pl.BlockSpec((EB, HP), lambda i: (i, 0)),
            pl.BlockSpec((EB, FE), lambda i: (i, 0)),
            _full((FE, H)), _full((H, L)), _full((1, L)),
        ],
        out_specs=pl.BlockSpec((EB, WS), lambda i: (i, 0)),
        out_shape=jax.ShapeDtypeStruct((E, WS), F32),
    )(ga, gb, ea, weT, w2T, b2)


def _k6_body(ga_r, gb_r, ea_r, ea1_r, weaT_r, wea1T_r, w2T_r, b2_r, out_r):
    ea1 = ea1_r[...][:, :L]
    h = jax.nn.relu(_unpack_bf16(ga_r[...]) + _unpack_bf16(gb_r[...])
                    + _dot(ea_r[...], weaT_r[...])
                    + _dot(ea1, wea1T_r[...]))
    msg = _dot(h, w2T_r[...]) + b2_r[...]
    out_r[...] = jnp.concatenate([msg, jnp.zeros((EB, WS - L), F32)], axis=1)


def _k6(ga, gb, ea, ea1, weaT, wea1T, w2T, b2):
    return pl.pallas_call(
        _k6_body,
        grid=(E // EB,),
        in_specs=[
            pl.BlockSpec((EB, HP), lambda i: (i, 0)),
            pl.BlockSpec((EB, HP), lambda i: (i, 0)),
            pl.BlockSpec((EB, FE), lambda i: (i, 0)),
            pl.BlockSpec((EB, WS), lambda i: (i, 0)),
            _full((FE, H)), _full((L, H)), _full((H, L)), _full((1, L)),
        ],
        out_specs=pl.BlockSpec((EB, WS), lambda i: (i, 0)),
        out_shape=jax.ShapeDtypeStruct((E, WS), F32),
    )(ga, gb, ea, ea1, weaT, wea1T, w2T, b2)


def _k4_body(x_r, h0_r, sums_r, wihx_r, wiha_r, bih_r, whh_r, bhh_r,
             w2sx_r, w2sh_r, b2s_r, w2dx_r, w2dh_r,
             h1_r, a2_r, b2_r):
    s = sums_r[0] + sums_r[1]
    agg = s[:, :L] / jnp.maximum(s[:, L:L + 1], 1.0)
    x = x_r[...]
    h0 = h0_r[...]
    gi = _dot(x, wihx_r[...]) + _dot(agg, wiha_r[...]) + bih_r[...]
    gh = _dot(h0, whh_r[...]) + bhh_r[...]
    r = jax.nn.sigmoid(gi[:, :R] + gh[:, :R])
    z = jax.nn.sigmoid(gi[:, R:2 * R] + gh[:, R:2 * R])
    n = jnp.tanh(gi[:, 2 * R:] + r * gh[:, 2 * R:])
    h1 = (1.0 - z) * n + z * h0
    h1_r[...] = h1
    a2_r[...] = _pack_bf16(_dot(x, w2sx_r[...]) + _dot(h1, w2sh_r[...])
                           + b2s_r[...])
    b2_r[...] = _pack_bf16(_dot(x, w2dx_r[...]) + _dot(h1, w2dh_r[...]))


def _k4(x, h0, sums, wihxT, wihaT, bih, whhT, bhh,
        w2sxT, w2shT, b2s, w2dxT, w2dhT):
    return pl.pallas_call(
        _k4_body,
        grid=(N // RB,),
        in_specs=[
            pl.BlockSpec((RB, F), lambda i: (i, 0)),
            pl.BlockSpec((RB, R), lambda i: (i, 0)),
            pl.BlockSpec((NC, RB, WS), lambda i: (0, i, 0)),
            _full((F, 3 * R)), _full((L, 3 * R)), _full((1, 3 * R)),
            _full((R, 3 * R)), _full((1, 3 * R)),
            _full((F, H)), _full((R, H)), _full((1, H)),
            _full((F, H)), _full((R, H)),
        ],
        out_specs=[
            pl.BlockSpec((RB, R), lambda i: (i, 0)),
            pl.BlockSpec((RB, HP), lambda i: (i, 0)),
            pl.BlockSpec((RB, HP), lambda i: (i, 0)),
        ],
        out_shape=[
            jax.ShapeDtypeStruct((N, R), F32),
            jax.ShapeDtypeStruct((N, HP), F32),
            jax.ShapeDtypeStruct((N, HP), F32),
        ],
    )(x, h0, sums, wihxT, wihaT, bih, whhT, bhh,
      w2sxT, w2shT, b2s, w2dxT, w2dhT)


def _k8_body(x_r, h1_r, sums_r, cnts_r, wx_r, wh_r, wa_r, b1_r, w2_r, b2_r,
             out_r):
    cnt = cnts_r[0, :, L:L + 1] + cnts_r[1, :, L:L + 1]
    agg = (sums_r[0] + sums_r[1])[:, :L] / jnp.maximum(cnt, 1.0)
    hh = jax.nn.relu(_dot(x_r[...], wx_r[...]) + _dot(h1_r[...], wh_r[...])
                     + _dot(agg, wa_r[...]) + b1_r[...])
    out_r[...] = _dot(hh, w2_r[...]) + b2_r[...]


def _k8(x, h1, sums, cnts, wxT, whT, waT, b1, w2T, b2):
    return pl.pallas_call(
        _k8_body,
        grid=(N // RB,),
        in_specs=[
            pl.BlockSpec((RB, F), lambda i: (i, 0)),
            pl.BlockSpec((RB, R), lambda i: (i, 0)),
            pl.BlockSpec((NC, RB, WS), lambda i: (0, i, 0)),
            pl.BlockSpec((NC, RB, WS), lambda i: (0, i, 0)),
            _full((F, H)), _full((R, H)), _full((L, H)), _full((1, H)),
            _full((H, O)), _full((1, O)),
        ],
        out_specs=pl.BlockSpec((RB, O), lambda i: (i, 0)),
        out_shape=jax.ShapeDtypeStruct((N, O), F32),
    )(x, h1, sums, cnts, wxT, whT, waT, b1, w2T, b2)


def kernel(x, edge_index, edge_attr, hidden, params):
    p = params
    row2d = edge_index[0].reshape(NW, CPW, C)
    col2d = edge_index[1].reshape(NW, CPW, C)
    h0 = hidden[0]

    zs = jnp.zeros((NP, WS), F32)

    w1 = p["e1_w1"]
    b1 = p["e1_b1"].reshape(1, H)
    a1, bb1 = _k0(x, w1[:, :F].T, w1[:, F:2 * F].T, b1)
    g1a, g1b = _sc_gather2(a1, bb1, row2d, col2d)
    ea1 = _k2(g1a, g1b, edge_attr, w1[:, 2 * F:].T, p["e1_w2"].T,
              p["e1_b2"].reshape(1, L))
    sums1 = _sc_scatter(ea1, edge_index[1], zs).reshape(NC, NP, WS)

    wih = p["w_ih"]
    w2 = p["e2_w1"]
    h1, a2, bb2 = _k4(
        x, h0, sums1,
        wih[:, :F].T, wih[:, F:].T, p["b_ih"].reshape(1, 3 * R),
        p["w_hh"].T, p["b_hh"].reshape(1, 3 * R),
        w2[:, :F].T, w2[:, F:256].T, p["e2_b1"].reshape(1, H),
        w2[:, 256:256 + F].T, w2[:, 256 + F:512].T,
    )
    g2a, g2b = _sc_gather2(a2, bb2, row2d, col2d)
    ea2 = _k6(g2a, g2b, edge_attr, ea1, w2[:, 512:512 + FE].T,
              w2[:, 512 + FE:].T, p["e2_w2"].T, p["e2_b2"].reshape(1, L))
    sums2 = _sc_scatter(ea2, edge_index[1], zs).reshape(NC, NP, WS)

    w3 = p["n2_w1"]
    out = _k8(x, h1, sums2, sums1,
              w3[:, :F].T, w3[:, F:256].T, w3[:, 256:].T,
              p["n2_b1"].reshape(1, H), p["n2_w2"].T,
              p["n2_b2"].reshape(1, O))
    return (out, h1[None])


# speedup vs baseline: 3.6696x; 1.0114x over previous
import functools

import jax
import jax.numpy as jnp
from jax import lax
from jax.experimental import pallas as pl
from jax.experimental.pallas import tpu as pltpu
from jax.experimental.pallas import tpu_sc as plsc

F32 = jnp.float32
BF16 = jnp.bfloat16

N = 10000
E = 320000
F = 128
FE = 16
H = 256
R = 128
L = 64
O = 128

NC = 2
NS = 16
NW = NC * NS
EPW = E // NW
C = 80
CPW = EPW // C
NP = 10240
NPT = NP // NS
WS = 128

_SC_MESH = plsc.VectorSubcoreMesh(core_axis_name="c", subcore_axis_name="s",
                                  num_cores=NC, num_subcores=NS)


def _dot(a, b):
    return jnp.dot(a, b, preferred_element_type=F32)


HP = H // 2


def _rne16(u):
    return u + jnp.uint32(0x7FFF) + ((u >> 16) & jnp.uint32(1))


def _pack_bf16(v):
    lo = _rne16(jax.lax.bitcast_convert_type(v[:, :HP], jnp.uint32))
    hi = _rne16(jax.lax.bitcast_convert_type(v[:, HP:], jnp.uint32))
    packed = (hi & jnp.uint32(0xFFFF0000)) | (lo >> 16)
    return jax.lax.bitcast_convert_type(packed, F32)


def _unpack_bf16(g):
    u = jax.lax.bitcast_convert_type(g, jnp.uint32)
    lo = jax.lax.bitcast_convert_type(u << 16, F32)
    hi = jax.lax.bitcast_convert_type(u & jnp.uint32(0xFFFF0000), F32)
    return jnp.concatenate([lo, hi], axis=1)


@functools.partial(
    pl.kernel,
    out_type=(
        jax.ShapeDtypeStruct((E, HP), F32),
        jax.ShapeDtypeStruct((E, HP), F32),
    ),
    mesh=_SC_MESH,
    scratch_types=[
        pltpu.VMEM((CPW, C), jnp.int32),
        pltpu.VMEM((CPW, C), jnp.int32),
        pltpu.VMEM((C, HP), F32),
        pltpu.VMEM((C, HP), F32),
        pltpu.VMEM((C, HP), F32),
        pltpu.VMEM((C, HP), F32),
        pltpu.SemaphoreType.DMA,
        pltpu.SemaphoreType.DMA,
        pltpu.SemaphoreType.DMA,
        pltpu.SemaphoreType.DMA,
        pltpu.SemaphoreType.DMA,
        pltpu.SemaphoreType.DMA,
        pltpu.SemaphoreType.DMA,
        pltpu.SemaphoreType.DMA,
    ],
)
def _sc_gather2(a_hbm, b_hbm, row_hbm, col_hbm, oa_hbm, ob_hbm,
                ridx, cidx, bufa0, bufb0, bufa1, bufb1,
                ga0, gb0, wa0, wb0, ga1, gb1, wa1, wb1):
    wid = lax.axis_index("s") * NC + lax.axis_index("c")
    pltpu.sync_copy(row_hbm.at[wid], ridx)
    pltpu.sync_copy(col_hbm.at[wid], cidx)
    bufa = (bufa0, bufa1)
    bufb = (bufb0, bufb1)
    ga = (ga0, ga1)
    gb = (gb0, gb1)
    wa = (wa0, wa1)
    wb = (wb0, wb1)

    def out_at(j):
        return pl.ds(wid * EPW + j * C, C)

    def step(i, _):
        for b in (0, 1):
            j = 2 * i + b
            @pl.when(i > 0)
            def _():
                pltpu.make_async_copy(bufa[b], oa_hbm.at[out_at(j)],
                                      wa[b]).wait()
                pltpu.make_async_copy(bufb[b], ob_hbm.at[out_at(j)],
                                      wb[b]).wait()
            pltpu.async_copy(a_hbm.at[ridx.at[j]], bufa[b], ga[b])
            pltpu.async_copy(b_hbm.at[cidx.at[j]], bufb[b], gb[b])
        for b in (0, 1):
            j = 2 * i + b
            pltpu.make_async_copy(a_hbm.at[ridx.at[j]], bufa[b], ga[b]).wait()
            pltpu.make_async_copy(b_hbm.at[cidx.at[j]], bufb[b], gb[b]).wait()
            pltpu.async_copy(bufa[b], oa_hbm.at[out_at(j)], wa[b])
            pltpu.async_copy(bufb[b], ob_hbm.at[out_at(j)], wb[b])
        return 0

    lax.fori_loop(0, CPW // 2, step, 0)
    for b in (0, 1):
        pltpu.make_async_copy(bufa[b], oa_hbm.at[out_at(0)], wa[b]).wait()
        pltpu.make_async_copy(bufb[b], ob_hbm.at[out_at(0)], wb[b]).wait()
    if CPW % 2:
        j = CPW - 1
        pltpu.sync_copy(a_hbm.at[ridx.at[j]], bufa0)
        pltpu.sync_copy(b_hbm.at[cidx.at[j]], bufb0)
        pltpu.sync_copy(bufa0, oa_hbm.at[out_at(j)])
        pltpu.sync_copy(bufb0, ob_hbm.at[out_at(j)])


def _make_sc_scatter(width):
    assert width == WS

    @functools.partial(
        pl.kernel,
        out_type=jax.ShapeDtypeStruct((NC * NP, width), F32),
        mesh=_SC_MESH,
        scratch_types=[
            pltpu.VMEM((EPW,), jnp.int32),
            pltpu.VMEM((C, width), F32),
            pltpu.VMEM((C, width), F32),
            pltpu.SemaphoreType.DMA,
            pltpu.SemaphoreType.DMA,
            pltpu.SemaphoreType.DMA,
            pltpu.SemaphoreType.DMA,
            pltpu.VMEM_SHARED((NP, width), F32),
        ],
    )
    def scatter(vals_hbm, colf_hbm, zs_hbm, sum_out, cidx, vals0, vals1,
                lv0, lv1, s0, s1, sum_sh):
        cid = lax.axis_index("c")
        sid = lax.axis_index("s")
        wid = sid * NC + cid
        vals = (vals0, vals1)
        lv = (lv0, lv1)
        sc = (s0, s1)
        pltpu.sync_copy(zs_hbm.at[pl.ds(0, C)], vals0)

        def zstep(t, _):
            pltpu.sync_copy(vals0, sum_sh.at[pl.ds(sid * NPT + t * C, C)])
            return 0

        lax.fori_loop(0, NPT // C, zstep, 0)
        pltpu.sync_copy(colf_hbm.at[pl.ds(wid * EPW, EPW)], cidx)
        plsc.subcore_barrier()

        def vals_at(j):
            return pl.ds(wid * EPW + j * C, C)

        def drain(b):
            for k in range(C // 16):
                pltpu.make_async_copy(vals[b].at[pl.ds(k * 16, 16)],
                                      sum_sh.at[pl.ds(0, 16)], sc[b]).wait()

        def fire(b, j):
            for k in range(C // 16):
                idx16 = cidx[pl.ds(j * C + k * 16, 16)]
                pltpu.async_copy(vals[b].at[pl.ds(k * 16, 16)],
                                 sum_sh.at[idx16], sc[b], add=True)

        def step(i, _):
            for b in (0, 1):
                @pl.when(i > 0)
                def _():
                    drain(b)
                pltpu.async_copy(vals_hbm.at[vals_at(2 * i + b)], vals[b],
                                 lv[b])
            for b in (0, 1):
                pltpu.make_async_copy(vals_hbm.at[vals_at(2 * i + b)],
                                      vals[b], lv[b]).wait()
                fire(b, 2 * i + b)
            return 0

        lax.fori_loop(0, CPW // 2, step, 0)
        for b in (0, 1):
            drain(b)
        if CPW % 2:
            j = CPW - 1
            pltpu.sync_copy(vals_hbm.at[vals_at(j)], vals0)
            fire(0, j)
            drain(0)
        plsc.subcore_barrier()

        def wstep(t, _):
            pltpu.sync_copy(sum_sh.at[pl.ds(sid * NPT + t * C, C)], vals0)
            pltpu.sync_copy(vals0,
                            sum_out.at[pl.ds(cid * NP + sid * NPT + t * C, C)])
            return 0

        lax.fori_loop(0, NPT // C, wstep, 0)

    return scatter


_sc_scatter = _make_sc_scatter(WS)


RB = 2000
EB = 8000


def _full(shape):
    return pl.BlockSpec(shape, lambda i: (0,) * len(shape))


def _k0_body(x_r, wsx_r, wdx_r, b1_r, a_r, b_r):
    x = x_r[...]
    a_r[...] = _pack_bf16(_dot(x, wsx_r[...]) + b1_r[...])
    b_r[...] = _pack_bf16(_dot(x, wdx_r[...]))


def _k0(x, wsxT, wdxT, b1):
    return pl.pallas_call(
        _k0_body,
        grid=(N // RB,),
        in_specs=[
            pl.BlockSpec((RB, F), lambda i: (i, 0)),
            _full((F, H)), _full((F, H)), _full((1, H)),
        ],
        out_specs=[pl.BlockSpec((RB, HP), lambda i: (i, 0))] * 2,
        out_shape=[jax.ShapeDtypeStruct((N, HP), F32)] * 2,
    )(x, wsxT, wdxT, b1)


def _k2_body(ga_r, gb_r, ea_r, weT_r, w2T_r, b2_r, out_r):
    h = jax.nn.relu(_unpack_bf16(ga_r[...]) + _unpack_bf16(gb_r[...])
                    + _dot(ea_r[...], weT_r[...]))
    msg = _dot(h, w2T_r[...]) + b2_r[...]
    out_r[...] = jnp.concatenate(
        [msg, jnp.ones((EB, FE), F32), jnp.zeros((EB, WS - L - FE), F32)],
        axis=1)


def _k2(ga, gb, ea, weT, w2T, b2):
    return pl.pallas_call(
        _k2_body,
        grid=(E // EB,),
        in_specs=[
            pl.BlockSpec((EB, HP), lambda i: (i, 0)),
            pl.BlockSpec((EB, HP), lambda i: (i, 0)),
            pl.BlockSpec((EB, FE), lambda i: (i, 0)),
            _full((FE, H)), _full((H, L)), _full((1, L)),
        ],
        out_specs=pl.BlockSpec((EB, WS), lambda i: (i, 0)),
        out_shape=jax.ShapeDtypeStruct((E, WS), F32),
    )(ga, gb, ea, weT, w2T, b2)


def _k6_body(ga_r, gb_r, ea_r, ea1_r, weaT_r, wea1T_r, w2T_r, b2_r, out_r):
    ea1 = ea1_r[...][:, :L]
    h = jax.nn.relu(_unpack_bf16(ga_r[...]) + _unpack_bf16(gb_r[...])
                    + _dot(ea_r[...], weaT_r[...])
                    + _dot(ea1, wea1T_r[...]))
    msg = _dot(h, w2T_r[...]) + b2_r[...]
    out_r[...] = jnp.concatenate([msg, jnp.zeros((EB, WS - L), F32)], axis=1)


def _k6(ga, gb, ea, ea1, weaT, wea1T, w2T, b2):
    return pl.pallas_call(
        _k6_body,
        grid=(E // EB,),
        in_specs=[
            pl.BlockSpec((EB, HP), lambda i: (i, 0)),
            pl.BlockSpec((EB, HP), lambda i: (i, 0)),
            pl.BlockSpec((EB, FE), lambda i: (i, 0)),
            pl.BlockSpec((EB, WS), lambda i: (i, 0)),
            _full((FE, H)), _full((L, H)), _full((H, L)), _full((1, L)),
        ],
        out_specs=pl.BlockSpec((EB, WS), lambda i: (i, 0)),
        out_shape=jax.ShapeDtypeStruct((E, WS), F32),
    )(ga, gb, ea, ea1, weaT, wea1T, w2T, b2)


def _k4_body(x_r, h0_r, sums_r, wihx_r, wiha_r, bih_r, whh_r, bhh_r,
             w2sx_r, w2sh_r, b2s_r, w2dx_r, w2dh_r,
             h1_r, a2_r, b2_r):
    s = sums_r[0] + sums_r[1]
    agg = s[:, :L] / jnp.maximum(s[:, L:L + 1], 1.0)
    x = x_r[...]
    h0 = h0_r[...]
    gi = _dot(x, wihx_r[...]) + _dot(agg, wiha_r[...]) + bih_r[...]
    gh = _dot(h0, whh_r[...]) + bhh_r[...]
    r = jax.nn.sigmoid(gi[:, :R] + gh[:, :R])
    z = jax.nn.sigmoid(gi[:, R:2 * R] + gh[:, R:2 * R])
    n = jnp.tanh(gi[:, 2 * R:] + r * gh[:, 2 * R:])
    h1 = (1.0 - z) * n + z * h0
    h1_r[...] = h1
    a2_r[...] = _pack_bf16(_dot(x, w2sx_r[...]) + _dot(h1, w2sh_r[...])
                           + b2s_r[...])
    b2_r[...] = _pack_bf16(_dot(x, w2dx_r[...]) + _dot(h1, w2dh_r[...]))


def _k4(x, h0, sums, wihxT, wihaT, bih, whhT, bhh,
        w2sxT, w2shT, b2s, w2dxT, w2dhT):
    return pl.pallas_call(
        _k4_body,
        grid=(N // RB,),
        in_specs=[
            pl.BlockSpec((RB, F), lambda i: (i, 0)),
            pl.BlockSpec((RB, R), lambda i: (i, 0)),
            pl.BlockSpec((NC, RB, WS), lambda i: (0, i, 0)),
            _full((F, 3 * R)), _full((L, 3 * R)), _full((1, 3 * R)),
            _full((R, 3 * R)), _full((1, 3 * R)),
            _full((F, H)), _full((R, H)), _full((1, H)),
            _full((F, H)), _full((R, H)),
        ],
        out_specs=[
            pl.BlockSpec((RB, R), lambda i: (i, 0)),
            pl.BlockSpec((RB, HP), lambda i: (i, 0)),
            pl.BlockSpec((RB, HP), lambda i: (i, 0)),
        ],
        out_shape=[
            jax.ShapeDtypeStruct((N, R), F32),
            jax.ShapeDtypeStruct((N, HP), F32),
            jax.ShapeDtypeStruct((N, HP), F32),
        ],
    )(x, h0, sums, wihxT, wihaT, bih, whhT, bhh,
      w2sxT, w2shT, b2s, w2dxT, w2dhT)


def _k8_body(x_r, h1_r, sums_r, cnts_r, wx_r, wh_r, wa_r, b1_r, w2_r, b2_r,
             out_r):
    cnt = cnts_r[0, :, L:L + 1] + cnts_r[1, :, L:L + 1]
    agg = (sums_r[0] + sums_r[1])[:, :L] / jnp.maximum(cnt, 1.0)
    hh = jax.nn.relu(_dot(x_r[...], wx_r[...]) + _dot(h1_r[...], wh_r[...])
                     + _dot(agg, wa_r[...]) + b1_r[...])
    out_r[...] = _dot(hh, w2_r[...]) + b2_r[...]


def _k8(x, h1, sums, cnts, wxT, whT, waT, b1, w2T, b2):
    return pl.pallas_call(
        _k8_body,
        grid=(N // RB,),
        in_specs=[
            pl.BlockSpec((RB, F), lambda i: (i, 0)),
            pl.BlockSpec((RB, R), lambda i: (i, 0)),
            pl.BlockSpec((NC, RB, WS), lambda i: (0, i, 0)),
            pl.BlockSpec((NC, RB, WS), lambda i: (0, i, 0)),
            _full((F, H)), _full((R, H)), _full((L, H)), _full((1, H)),
            _full((H, O)), _full((1, O)),
        ],
        out_specs=pl.BlockSpec((RB, O), lambda i: (i, 0)),
        out_shape=jax.ShapeDtypeStruct((N, O), F32),
    )(x, h1, sums, cnts, wxT, whT, waT, b1, w2T, b2)


def kernel(x, edge_index, edge_attr, hidden, params):
    p = params
    row2d = edge_index[0].reshape(NW, CPW, C)
    col2d = edge_index[1].reshape(NW, CPW, C)
    h0 = hidden[0]

    zs = jnp.zeros((NP, WS), F32)

    w1 = p["e1_w1"]
    b1 = p["e1_b1"].reshape(1, H)
    a1, bb1 = _k0(x, w1[:, :F].T, w1[:, F:2 * F].T, b1)
    g1a, g1b = _sc_gather2(a1, bb1, row2d, col2d)
    ea1 = _k2(g1a, g1b, edge_attr, w1[:, 2 * F:].T, p["e1_w2"].T,
              p["e1_b2"].reshape(1, L))
    sums1 = _sc_scatter(ea1, edge_index[1], zs).reshape(NC, NP, WS)

    wih = p["w_ih"]
    w2 = p["e2_w1"]
    h1, a2, bb2 = _k4(
        x, h0, sums1,
        wih[:, :F].T, wih[:, F:].T, p["b_ih"].reshape(1, 3 * R),
        p["w_hh"].T, p["b_hh"].reshape(1, 3 * R),
        w2[:, :F].T, w2[:, F:256].T, p["e2_b1"].reshape(1, H),
        w2[:, 256:256 + F].T, w2[:, 256 + F:512].T,
    )
    g2a, g2b = _sc_gather2(a2, bb2, row2d, col2d)
    ea2 = _k6(g2a, g2b, edge_attr, ea1, w2[:, 512:512 + FE].T,
              w2[:, 512 + FE:].T, p["e2_w2"].T, p["e2_b2"].reshape(1, L))
    sums2 = _sc_scatter(ea2, edge_index[1], zs).reshape(NC, NP, WS)

    w3 = p["n2_w1"]
    out = _k8(x, h1, sums2, sums1,
              w3[:, :F].T, w3[:, F:256].T, w3[:, 256:].T,
              p["n2_b1"].reshape(1, H), p["n2_w2"].T,
              p["n2_b2"].reshape(1, O))
    return (out, h1[None])
